# Initial kernel scaffold; baseline (speedup 1.0000x reference)
import jax
import jax.numpy as jnp
from jax.experimental import pallas as pl


def kernel(inputs, embeddings, is_training):
    raise NotImplementedError("write your pallas kernel here")


# speedup vs baseline: 1.3012x; 1.3012x over previous
import functools

import jax
import jax.numpy as jnp
from jax.experimental import pallas as pl
from jax.experimental.pallas import tpu as pltpu

EMBEDDING_DIM = 32
NUM_EMBEDDINGS = 8192
COMMITMENT_COST = 0.25

TN = 512
TK = 1024


def _pass1_body(x_ref, e_ref, d_ref, idx_ref, q_ref, loss_ref, min_ref):
    k = pl.program_id(1)
    n = pl.program_id(0)
    nk = pl.num_programs(1)
    nn = pl.num_programs(0)

    x = x_ref[...]
    e = e_ref[...]
    x2 = jnp.sum(x * x, axis=1, keepdims=True)
    e2 = jnp.sum(e * e, axis=0, keepdims=True)
    xe = jnp.dot(x, e, preferred_element_type=jnp.float32)
    d = x2 - 2.0 * xe + e2
    d_ref[...] = d

    local_idx = jnp.argmin(d, axis=1)
    local_min = jnp.min(d, axis=1, keepdims=True)
    glob_idx = (local_idx + k * TK)[:, None]

    lanes = jax.lax.broadcasted_iota(jnp.int32, d.shape, 1)
    onehot = (lanes == local_idx[:, None]).astype(jnp.float32)
    qcand = jax.lax.dot_general(
        onehot, e, (((1,), (1,)), ((), ())),
        preferred_element_type=jnp.float32)

    @pl.when(k == 0)
    def _init():
        min_ref[...] = local_min
        idx_ref[...] = glob_idx
        q_ref[...] = qcand

    @pl.when(k > 0)
    def _update():
        better = local_min < min_ref[...]
        min_ref[...] = jnp.where(better, local_min, min_ref[...])
        idx_ref[...] = jnp.where(better, glob_idx, idx_ref[...])
        q_ref[...] = jnp.where(better, qcand, q_ref[...])

    @pl.when(jnp.logical_and(n == 0, k == 0))
    def _loss_init():
        loss_ref[...] = jnp.zeros_like(loss_ref)

    @pl.when(k == nk - 1)
    def _finalize():
        q = q_ref[...]
        diff = q - x
        loss_ref[...] += jnp.sum(diff * diff).reshape(1, 1)
        q_ref[...] = x + (q - x)

        @pl.when(n == nn - 1)
        def _loss_final():
            total = jnp.float32(q.shape[1]) * jnp.float32(TN) * nn
            loss_ref[...] = loss_ref[...] * (COMMITMENT_COST / total)


def _pass2_body(idx_ref, enc_ref, perp_ref, counts_ref):
    n = pl.program_id(0)
    k = pl.program_id(1)
    nk = pl.num_programs(1)
    nn = pl.num_programs(0)

    idx = idx_ref[...]
    lanes = jax.lax.broadcasted_iota(jnp.int32, (TN, TK), 1) + k * TK
    enc = (lanes == idx).astype(jnp.float32)
    enc_ref[...] = enc

    colsum = jnp.sum(enc, axis=0, keepdims=True)

    @pl.when(n == 0)
    def _init():
        counts_ref[:, pl.ds(k * TK, TK)] = colsum

    @pl.when(n > 0)
    def _acc():
        counts_ref[:, pl.ds(k * TK, TK)] += colsum

    @pl.when(jnp.logical_and(n == nn - 1, k == nk - 1))
    def _final():
        total = jnp.float32(TN) * nn
        avg = counts_ref[...] / total
        ent = jnp.sum(avg * jnp.log(avg + 1e-10))
        perp_ref[...] = jnp.exp(-ent).reshape(1, 1)


def kernel(inputs, embeddings, is_training):
    del is_training
    D = embeddings.shape[0]
    K = embeddings.shape[1]
    flat = jnp.reshape(inputs, (-1, D))
    N = flat.shape[0]
    nn = N // TN
    nk = K // TK

    distances, idx2d, quant, loss11 = pl.pallas_call(
        _pass1_body,
        grid=(nn, nk),
        in_specs=[
            pl.BlockSpec((TN, D), lambda n, k: (n, 0)),
            pl.BlockSpec((D, TK), lambda n, k: (0, k)),
        ],
        out_specs=[
            pl.BlockSpec((TN, TK), lambda n, k: (n, k)),
            pl.BlockSpec((TN, 1), lambda n, k: (n, 0)),
            pl.BlockSpec((TN, D), lambda n, k: (n, 0)),
            pl.BlockSpec((1, 1), lambda n, k: (0, 0)),
        ],
        out_shape=[
            jax.ShapeDtypeStruct((N, K), jnp.float32),
            jax.ShapeDtypeStruct((N, 1), jnp.int32),
            jax.ShapeDtypeStruct((N, D), jnp.float32),
            jax.ShapeDtypeStruct((1, 1), jnp.float32),
        ],
        scratch_shapes=[pltpu.VMEM((TN, 1), jnp.float32)],
    )(flat, embeddings)

    encodings, perp11 = pl.pallas_call(
        _pass2_body,
        grid=(nn, nk),
        in_specs=[pl.BlockSpec((TN, 1), lambda n, k: (n, 0))],
        out_specs=[
            pl.BlockSpec((TN, TK), lambda n, k: (n, k)),
            pl.BlockSpec((1, 1), lambda n, k: (0, 0)),
        ],
        out_shape=[
            jax.ShapeDtypeStruct((N, K), jnp.float32),
            jax.ShapeDtypeStruct((1, 1), jnp.float32),
        ],
        scratch_shapes=[pltpu.VMEM((1, K), jnp.float32)],
    )(idx2d)

    quantized = jnp.reshape(quant, inputs.shape)
    encoding_indices = jnp.reshape(idx2d, inputs.shape[:-1])
    loss = loss11[0, 0]
    perplexity = perp11[0, 0]
    return (quantized, loss, perplexity, encodings, encoding_indices, distances)


# speedup vs baseline: 1.4576x; 1.1201x over previous
import jax
import jax.numpy as jnp
from jax.experimental import pallas as pl
from jax.experimental.pallas import tpu as pltpu

EMBEDDING_DIM = 32
NUM_EMBEDDINGS = 8192
COMMITMENT_COST = 0.25

TN = 512
TK = 1024


def _pass1_body(x_ref, e_ref, d_ref, idx_ref, min_ref, x2_ref):
    k = pl.program_id(1)
    tk = e_ref.shape[1]

    x = x_ref[...]
    e = e_ref[...]

    @pl.when(k == 0)
    def _x2():
        x2_ref[...] = jnp.sum(x * x, axis=1, keepdims=True)

    x2 = x2_ref[...]
    e2 = jnp.sum(e * e, axis=0, keepdims=True)
    xe = jnp.dot(x, e, preferred_element_type=jnp.float32)
    d = x2 - 2.0 * xe + e2
    d_ref[...] = d

    local_idx = jnp.argmin(d, axis=1)
    local_min = jnp.min(d, axis=1, keepdims=True)
    glob_idx = (local_idx + k * tk)[:, None]

    @pl.when(k == 0)
    def _init():
        min_ref[...] = local_min
        idx_ref[...] = glob_idx

    @pl.when(k > 0)
    def _update():
        better = local_min < min_ref[...]
        min_ref[...] = jnp.where(better, local_min, min_ref[...])
        idx_ref[...] = jnp.where(better, glob_idx, idx_ref[...])


def _pass2_body(idx_ref, e_ref, x_ref, enc_ref, q_ref, loss_ref, perp_ref,
                counts_ref):
    n = pl.program_id(0)
    k = pl.program_id(1)
    nk = pl.num_programs(1)
    nn = pl.num_programs(0)

    idx_local = idx_ref[...] - k * TK
    lanes = jax.lax.broadcasted_iota(jnp.int32, (TN, TK), 1)
    enc = (lanes == idx_local).astype(jnp.float32)
    enc_ref[...] = enc

    colsum = jnp.sum(enc, axis=0, keepdims=True)

    qpart = jax.lax.dot_general(
        enc, e_ref[...], (((1,), (1,)), ((), ())),
        preferred_element_type=jnp.float32)

    @pl.when(k == 0)
    def _init():
        counts_ref[:, pl.ds(k * TK, TK)] = colsum
        q_ref[...] = qpart

    @pl.when(k > 0)
    def _acc():
        counts_ref[:, pl.ds(k * TK, TK)] += colsum
        q_ref[...] += qpart

    @pl.when(jnp.logical_and(n == 0, k == 0))
    def _loss_init():
        loss_ref[...] = jnp.zeros_like(loss_ref)

    @pl.when(k == nk - 1)
    def _finalize():
        x = x_ref[...]
        q = q_ref[...]
        diff = q - x
        loss_ref[...] += jnp.sum(diff * diff).reshape(1, 1)
        q_ref[...] = x + (q - x)

        @pl.when(n == nn - 1)
        def _final():
            total_el = jnp.float32(x.shape[1]) * jnp.float32(TN) * nn
            loss_ref[...] = loss_ref[...] * (COMMITMENT_COST / total_el)
            rows = jnp.float32(TN) * nn
            avg = counts_ref[...] / rows
            ent = jnp.sum(avg * jnp.log(avg + 1e-10))
            perp_ref[...] = jnp.exp(-ent).reshape(1, 1)


def kernel(inputs, embeddings, is_training):
    del is_training
    D = embeddings.shape[0]
    K = embeddings.shape[1]
    flat = jnp.reshape(inputs, (-1, D))
    N = flat.shape[0]
    nn = N // TN
    nk = K // TK

    distances, idx2d = pl.pallas_call(
        _pass1_body,
        grid=(nn, nk),
        in_specs=[
            pl.BlockSpec((TN, D), lambda n, k: (n, 0)),
            pl.BlockSpec((D, TK), lambda n, k: (0, k)),
        ],
        out_specs=[
            pl.BlockSpec((TN, TK), lambda n, k: (n, k)),
            pl.BlockSpec((TN, 1), lambda n, k: (n, 0)),
        ],
        out_shape=[
            jax.ShapeDtypeStruct((N, K), jnp.float32),
            jax.ShapeDtypeStruct((N, 1), jnp.int32),
        ],
        scratch_shapes=[
            pltpu.VMEM((TN, 1), jnp.float32),
            pltpu.VMEM((TN, 1), jnp.float32),
        ],
    )(flat, embeddings)

    encodings, quant, loss11, perp11 = pl.pallas_call(
        _pass2_body,
        grid=(nn, nk),
        in_specs=[
            pl.BlockSpec((TN, 1), lambda n, k: (n, 0)),
            pl.BlockSpec((D, TK), lambda n, k: (0, k)),
            pl.BlockSpec((TN, D), lambda n, k: (n, 0)),
        ],
        out_specs=[
            pl.BlockSpec((TN, TK), lambda n, k: (n, k)),
            pl.BlockSpec((TN, D), lambda n, k: (n, 0)),
            pl.BlockSpec((1, 1), lambda n, k: (0, 0)),
            pl.BlockSpec((1, 1), lambda n, k: (0, 0)),
        ],
        out_shape=[
            jax.ShapeDtypeStruct((N, K), jnp.float32),
            jax.ShapeDtypeStruct((N, D), jnp.float32),
            jax.ShapeDtypeStruct((1, 1), jnp.float32),
            jax.ShapeDtypeStruct((1, 1), jnp.float32),
        ],
        scratch_shapes=[pltpu.VMEM((1, K), jnp.float32)],
    )(idx2d, embeddings, flat)

    quantized = jnp.reshape(quant, inputs.shape)
    encoding_indices = jnp.reshape(idx2d, inputs.shape[:-1])
    loss = loss11[0, 0]
    perplexity = perp11[0, 0]
    return (quantized, loss, perplexity, encodings, encoding_indices, distances)


# speedup vs baseline: 1.9362x; 1.3284x over previous
import jax
import jax.numpy as jnp
from jax.experimental import pallas as pl
from jax.experimental.pallas import tpu as pltpu

EMBEDDING_DIM = 32
NUM_EMBEDDINGS = 8192
COMMITMENT_COST = 0.25

TN = 512
TK = 2048


def _pass1_body(x_ref, e_ref, d_ref, idx_ref, min_ref, x2_ref):
    k = pl.program_id(1)
    tk = e_ref.shape[1]

    x = x_ref[...]
    e = e_ref[...]

    @pl.when(k == 0)
    def _x2():
        x2_ref[...] = jnp.sum(x * x, axis=1, keepdims=True)

    x2 = x2_ref[...]
    e2 = jnp.sum(e * e, axis=0, keepdims=True)
    xe = jnp.dot(x, e, preferred_element_type=jnp.float32)
    d = x2 - 2.0 * xe + e2
    d_ref[...] = d

    local_idx = jnp.argmin(d, axis=1)
    local_min = jnp.min(d, axis=1, keepdims=True)
    glob_idx = (local_idx + k * tk)[:, None]

    @pl.when(k == 0)
    def _init():
        min_ref[...] = local_min
        idx_ref[...] = glob_idx

    @pl.when(k > 0)
    def _update():
        better = local_min < min_ref[...]
        min_ref[...] = jnp.where(better, local_min, min_ref[...])
        idx_ref[...] = jnp.where(better, glob_idx, idx_ref[...])


def _pass2_body(idx_ref, e_ref, x_ref, enc_ref, q_ref, loss_ref, perp_ref,
                counts_ref):
    n = pl.program_id(0)
    k = pl.program_id(1)
    nk = pl.num_programs(1)
    nn = pl.num_programs(0)

    idx_local = idx_ref[...] - k * TK
    lanes = jax.lax.broadcasted_iota(jnp.int32, (TN, TK), 1)
    enc = (lanes == idx_local).astype(jnp.float32)
    enc_ref[...] = enc

    colsum = jnp.sum(enc, axis=0, keepdims=True)

    qpart = jax.lax.dot_general(
        enc, e_ref[...], (((1,), (1,)), ((), ())),
        preferred_element_type=jnp.float32)

    @pl.when(k == 0)
    def _init():
        counts_ref[:, pl.ds(k * TK, TK)] = colsum
        q_ref[...] = qpart

    @pl.when(k > 0)
    def _acc():
        counts_ref[:, pl.ds(k * TK, TK)] += colsum
        q_ref[...] += qpart

    @pl.when(jnp.logical_and(n == 0, k == 0))
    def _loss_init():
        loss_ref[...] = jnp.zeros_like(loss_ref)

    @pl.when(k == nk - 1)
    def _finalize():
        x = x_ref[...]
        q = q_ref[...]
        diff = q - x
        loss_ref[...] += jnp.sum(diff * diff).reshape(1, 1)
        q_ref[...] = x + (q - x)

        @pl.when(n == nn - 1)
        def _final():
            total_el = jnp.float32(x.shape[1]) * jnp.float32(TN) * nn
            loss_ref[...] = loss_ref[...] * (COMMITMENT_COST / total_el)
            rows = jnp.float32(TN) * nn
            avg = counts_ref[...] / rows
            ent = jnp.sum(avg * jnp.log(avg + 1e-10))
            perp_ref[...] = jnp.exp(-ent).reshape(1, 1)


def kernel(inputs, embeddings, is_training):
    del is_training
    D = embeddings.shape[0]
    K = embeddings.shape[1]
    flat = jnp.reshape(inputs, (-1, D))
    N = flat.shape[0]
    nn = N // TN
    nk = K // TK

    distances, idx2d = pl.pallas_call(
        _pass1_body,
        grid=(nn, nk),
        in_specs=[
            pl.BlockSpec((TN, D), lambda n, k: (n, 0)),
            pl.BlockSpec((D, TK), lambda n, k: (0, k)),
        ],
        out_specs=[
            pl.BlockSpec((TN, TK), lambda n, k: (n, k)),
            pl.BlockSpec((TN, 1), lambda n, k: (n, 0)),
        ],
        out_shape=[
            jax.ShapeDtypeStruct((N, K), jnp.float32),
            jax.ShapeDtypeStruct((N, 1), jnp.int32),
        ],
        scratch_shapes=[
            pltpu.VMEM((TN, 1), jnp.float32),
            pltpu.VMEM((TN, 1), jnp.float32),
        ],
    )(flat, embeddings)

    encodings, quant, loss11, perp11 = pl.pallas_call(
        _pass2_body,
        grid=(nn, nk),
        in_specs=[
            pl.BlockSpec((TN, 1), lambda n, k: (n, 0)),
            pl.BlockSpec((D, TK), lambda n, k: (0, k)),
            pl.BlockSpec((TN, D), lambda n, k: (n, 0)),
        ],
        out_specs=[
            pl.BlockSpec((TN, TK), lambda n, k: (n, k)),
            pl.BlockSpec((TN, D), lambda n, k: (n, 0)),
            pl.BlockSpec((1, 1), lambda n, k: (0, 0)),
            pl.BlockSpec((1, 1), lambda n, k: (0, 0)),
        ],
        out_shape=[
            jax.ShapeDtypeStruct((N, K), jnp.float32),
            jax.ShapeDtypeStruct((N, D), jnp.float32),
            jax.ShapeDtypeStruct((1, 1), jnp.float32),
            jax.ShapeDtypeStruct((1, 1), jnp.float32),
        ],
        scratch_shapes=[pltpu.VMEM((1, K), jnp.float32)],
    )(idx2d, embeddings, flat)

    quantized = jnp.reshape(quant, inputs.shape)
    encoding_indices = jnp.reshape(idx2d, inputs.shape[:-1])
    loss = loss11[0, 0]
    perplexity = perp11[0, 0]
    return (quantized, loss, perplexity, encodings, encoding_indices, distances)


# speedup vs baseline: 2.2329x; 1.1532x over previous
import jax
import jax.numpy as jnp
from jax.experimental import pallas as pl
from jax.experimental.pallas import tpu as pltpu

EMBEDDING_DIM = 32
NUM_EMBEDDINGS = 8192
COMMITMENT_COST = 0.25

TN = 512
TK = 4096


def _pass1_body(x_ref, e_ref, d_ref, idx_ref, min_ref, x2_ref):
    k = pl.program_id(1)
    tk = e_ref.shape[1]

    x = x_ref[...]
    e = e_ref[...]

    @pl.when(k == 0)
    def _x2():
        x2_ref[...] = jnp.sum(x * x, axis=1, keepdims=True)

    x2 = x2_ref[...]
    e2 = jnp.sum(e * e, axis=0, keepdims=True)
    xe = jnp.dot(x, e, preferred_element_type=jnp.float32)
    d = x2 - 2.0 * xe + e2
    d_ref[...] = d

    local_idx = jnp.argmin(d, axis=1)
    local_min = jnp.min(d, axis=1, keepdims=True)
    glob_idx = (local_idx + k * tk)[:, None]

    @pl.when(k == 0)
    def _init():
        min_ref[...] = local_min
        idx_ref[...] = glob_idx

    @pl.when(k > 0)
    def _update():
        better = local_min < min_ref[...]
        min_ref[...] = jnp.where(better, local_min, min_ref[...])
        idx_ref[...] = jnp.where(better, glob_idx, idx_ref[...])


def _pass2_body(idx_ref, e_ref, x_ref, enc_ref, q_ref, loss_ref, perp_ref,
                counts_ref):
    n = pl.program_id(0)
    k = pl.program_id(1)
    nk = pl.num_programs(1)
    nn = pl.num_programs(0)

    idx_local = idx_ref[...] - k * TK
    lanes = jax.lax.broadcasted_iota(jnp.int32, (TN, TK), 1)
    enc = (lanes == idx_local).astype(jnp.float32)
    enc_ref[...] = enc

    colsum = jnp.sum(enc, axis=0, keepdims=True)

    qpart = jax.lax.dot_general(
        enc, e_ref[...], (((1,), (1,)), ((), ())),
        preferred_element_type=jnp.float32)

    @pl.when(k == 0)
    def _init():
        counts_ref[:, pl.ds(k * TK, TK)] = colsum
        q_ref[...] = qpart

    @pl.when(k > 0)
    def _acc():
        counts_ref[:, pl.ds(k * TK, TK)] += colsum
        q_ref[...] += qpart

    @pl.when(jnp.logical_and(n == 0, k == 0))
    def _loss_init():
        loss_ref[...] = jnp.zeros_like(loss_ref)

    @pl.when(k == nk - 1)
    def _finalize():
        x = x_ref[...]
        q = q_ref[...]
        diff = q - x
        loss_ref[...] += jnp.sum(diff * diff).reshape(1, 1)
        q_ref[...] = x + (q - x)

        @pl.when(n == nn - 1)
        def _final():
            total_el = jnp.float32(x.shape[1]) * jnp.float32(TN) * nn
            loss_ref[...] = loss_ref[...] * (COMMITMENT_COST / total_el)
            rows = jnp.float32(TN) * nn
            avg = counts_ref[...] / rows
            ent = jnp.sum(avg * jnp.log(avg + 1e-10))
            perp_ref[...] = jnp.exp(-ent).reshape(1, 1)


def kernel(inputs, embeddings, is_training):
    del is_training
    D = embeddings.shape[0]
    K = embeddings.shape[1]
    flat = jnp.reshape(inputs, (-1, D))
    N = flat.shape[0]
    nn = N // TN
    nk = K // TK

    distances, idx2d = pl.pallas_call(
        _pass1_body,
        grid=(nn, nk),
        in_specs=[
            pl.BlockSpec((TN, D), lambda n, k: (n, 0)),
            pl.BlockSpec((D, TK), lambda n, k: (0, k)),
        ],
        out_specs=[
            pl.BlockSpec((TN, TK), lambda n, k: (n, k)),
            pl.BlockSpec((TN, 1), lambda n, k: (n, 0)),
        ],
        out_shape=[
            jax.ShapeDtypeStruct((N, K), jnp.float32),
            jax.ShapeDtypeStruct((N, 1), jnp.int32),
        ],
        scratch_shapes=[
            pltpu.VMEM((TN, 1), jnp.float32),
            pltpu.VMEM((TN, 1), jnp.float32),
        ],
    )(flat, embeddings)

    encodings, quant, loss11, perp11 = pl.pallas_call(
        _pass2_body,
        grid=(nn, nk),
        in_specs=[
            pl.BlockSpec((TN, 1), lambda n, k: (n, 0)),
            pl.BlockSpec((D, TK), lambda n, k: (0, k)),
            pl.BlockSpec((TN, D), lambda n, k: (n, 0)),
        ],
        out_specs=[
            pl.BlockSpec((TN, TK), lambda n, k: (n, k)),
            pl.BlockSpec((TN, D), lambda n, k: (n, 0)),
            pl.BlockSpec((1, 1), lambda n, k: (0, 0)),
            pl.BlockSpec((1, 1), lambda n, k: (0, 0)),
        ],
        out_shape=[
            jax.ShapeDtypeStruct((N, K), jnp.float32),
            jax.ShapeDtypeStruct((N, D), jnp.float32),
            jax.ShapeDtypeStruct((1, 1), jnp.float32),
            jax.ShapeDtypeStruct((1, 1), jnp.float32),
        ],
        scratch_shapes=[pltpu.VMEM((1, K), jnp.float32)],
    )(idx2d, embeddings, flat)

    quantized = jnp.reshape(quant, inputs.shape)
    encoding_indices = jnp.reshape(idx2d, inputs.shape[:-1])
    loss = loss11[0, 0]
    perplexity = perp11[0, 0]
    return (quantized, loss, perplexity, encodings, encoding_indices, distances)


# speedup vs baseline: 2.4659x; 1.1044x over previous
import jax
import jax.numpy as jnp
from jax.experimental import pallas as pl
from jax.experimental.pallas import tpu as pltpu

EMBEDDING_DIM = 32
NUM_EMBEDDINGS = 8192
COMMITMENT_COST = 0.25

TN = 256


def _fused_body(x_ref, e_ref, d_ref, idx_ref, enc_ref, q_ref, loss_ref,
                perp_ref, counts_ref):
    n = pl.program_id(0)
    nn = pl.num_programs(0)
    K = e_ref.shape[1]

    x = x_ref[...]
    e = e_ref[...]
    x2 = jnp.sum(x * x, axis=1, keepdims=True)
    e2 = jnp.sum(e * e, axis=0, keepdims=True)
    xe = jnp.dot(x, e, preferred_element_type=jnp.float32)
    d = x2 - 2.0 * xe + e2
    d_ref[...] = d

    idx = jnp.argmin(d, axis=1)[:, None]
    idx_ref[...] = idx

    lanes = jax.lax.broadcasted_iota(jnp.int32, (TN, K), 1)
    enc = (lanes == idx).astype(jnp.float32)
    enc_ref[...] = enc

    colsum = jnp.sum(enc, axis=0, keepdims=True)

    q = jax.lax.dot_general(
        enc, e, (((1,), (1,)), ((), ())),
        preferred_element_type=jnp.float32)
    diff = q - x
    q_ref[...] = x + (q - x)

    @pl.when(n == 0)
    def _init():
        counts_ref[...] = colsum
        loss_ref[...] = jnp.sum(diff * diff).reshape(1, 1)

    @pl.when(n > 0)
    def _acc():
        counts_ref[...] += colsum
        loss_ref[...] += jnp.sum(diff * diff).reshape(1, 1)

    @pl.when(n == nn - 1)
    def _final():
        total_el = jnp.float32(x.shape[1]) * jnp.float32(TN) * nn
        loss_ref[...] = loss_ref[...] * (COMMITMENT_COST / total_el)
        rows = jnp.float32(TN) * nn
        avg = counts_ref[...] / rows
        ent = jnp.sum(avg * jnp.log(avg + 1e-10))
        perp_ref[...] = jnp.exp(-ent).reshape(1, 1)


def kernel(inputs, embeddings, is_training):
    del is_training
    D = embeddings.shape[0]
    K = embeddings.shape[1]
    flat = jnp.reshape(inputs, (-1, D))
    N = flat.shape[0]
    nn = N // TN

    distances, idx2d, encodings, quant, loss11, perp11 = pl.pallas_call(
        _fused_body,
        grid=(nn,),
        in_specs=[
            pl.BlockSpec((TN, D), lambda n: (n, 0)),
            pl.BlockSpec((D, K), lambda n: (0, 0)),
        ],
        out_specs=[
            pl.BlockSpec((TN, K), lambda n: (n, 0)),
            pl.BlockSpec((TN, 1), lambda n: (n, 0)),
            pl.BlockSpec((TN, K), lambda n: (n, 0)),
            pl.BlockSpec((TN, D), lambda n: (n, 0)),
            pl.BlockSpec((1, 1), lambda n: (0, 0)),
            pl.BlockSpec((1, 1), lambda n: (0, 0)),
        ],
        out_shape=[
            jax.ShapeDtypeStruct((N, K), jnp.float32),
            jax.ShapeDtypeStruct((N, 1), jnp.int32),
            jax.ShapeDtypeStruct((N, K), jnp.float32),
            jax.ShapeDtypeStruct((N, D), jnp.float32),
            jax.ShapeDtypeStruct((1, 1), jnp.float32),
            jax.ShapeDtypeStruct((1, 1), jnp.float32),
        ],
        scratch_shapes=[pltpu.VMEM((1, K), jnp.float32)],
    )(flat, embeddings)

    quantized = jnp.reshape(quant, inputs.shape)
    encoding_indices = jnp.reshape(idx2d, inputs.shape[:-1])
    loss = loss11[0, 0]
    perplexity = perp11[0, 0]
    return (quantized, loss, perplexity, encodings, encoding_indices, distances)
